# baseline (speedup 1.0000x reference)
import functools
import jax
import jax.numpy as jnp
from jax import lax
from jax.experimental import pallas as pl
from jax.experimental.pallas import tpu as pltpu
from jax.experimental.pallas import tpu_sc as plsc

NC = 2
NS = 16
NW = NC * NS
LANES = 16
BATCH = 16384
EMB = 64
PADW = 128
BPW = BATCH // NW
CHUNK = 128
NCHUNK = BPW // CHUNK
GROUPS = CHUNK // LANES
TBLK = 512


def _transpose_body(in_ref, out_ref):
    for t in range(TBLK // PADW):
        sub0 = in_ref[:, pl.ds(t * PADW, EMB)]
        sub1 = in_ref[:, pl.ds(t * PADW + EMB, EMB)]
        osl = pl.ds(t * EMB, EMB)
        out_ref[osl, pl.ds(0, EMB)] = jnp.swapaxes(sub0, 0, 1)
        out_ref[osl, pl.ds(EMB, EMB)] = jnp.swapaxes(sub1, 0, 1)


def _relayout(table_t, n_rows):
    grid = -(-n_rows // TBLK)
    out_rows = -(-n_rows // PADW) * EMB
    return pl.pallas_call(
        _transpose_body,
        grid=(grid,),
        in_specs=[pl.BlockSpec((EMB, TBLK), lambda i: (0, i))],
        out_specs=pl.BlockSpec((TBLK // 2, PADW), lambda i: (i, 0)),
        out_shape=jax.ShapeDtypeStruct((out_rows, PADW), jnp.float32),
    )(table_t)


def _body(user_hbm, movie_hbm, uemb_hbm, memb_hbm, ubias_hbm, mbias_hbm,
          gbias_hbm, out_hbm,
          uorig_v, morig_v, upack_v, mpack_v, ubrow_v, mbrow_v,
          urows_v, mrows_v, ubrows_v, mbrows_v, gb_v, out_v, buf_v, sem):
    wid = lax.axis_index("s") * NC + lax.axis_index("c")
    base = wid * BPW

    for j in range(NCHUNK):
        pltpu.sync_copy(user_hbm.at[pl.ds(base + j * CHUNK, CHUNK)],
                        uorig_v.at[j])
        pltpu.sync_copy(movie_hbm.at[pl.ds(base + j * CHUNK, CHUNK)],
                        morig_v.at[j])
    pltpu.sync_copy(gbias_hbm, gb_v.at[pl.ds(0, 1)])

    for j in range(NCHUNK):
        for h in range(GROUPS):
            sl = pl.ds(h * LANES, LANES)
            u = uorig_v[j, sl]
            m = morig_v[j, sl]
            ub = lax.shift_right_logical(u, 7)
            mb = lax.shift_right_logical(m, 7)
            upack_v[j, sl] = lax.bitwise_or(
                lax.shift_left(ub, 6), lax.bitwise_and(u, 63))
            mpack_v[j, sl] = lax.bitwise_or(
                lax.shift_left(mb, 6), lax.bitwise_and(m, 63))
            ubrow_v[j, sl] = ub
            mbrow_v[j, sl] = mb

    gb = gb_v[pl.ds(0, LANES)][0]
    iota = jax.lax.iota(jnp.int32, LANES)
    base_idx = iota * (LANES + 1)

    def chunk_body(c, carry):
        cu = pltpu.async_copy(uemb_hbm.at[upack_v.at[c]], urows_v, sem)
        cm = pltpu.async_copy(memb_hbm.at[mpack_v.at[c]], mrows_v, sem)
        cub = pltpu.async_copy(ubias_hbm.at[ubrow_v.at[c]], ubrows_v, sem)
        cmb = pltpu.async_copy(mbias_hbm.at[mbrow_v.at[c]], mbrows_v, sem)
        cu.wait()
        cm.wait()
        cub.wait()
        cmb.wait()

        for g in range(GROUPS):
            rbase = g * LANES
            sl = pl.ds(rbase, LANES)
            upar = lax.bitwise_and(
                lax.shift_right_logical(uorig_v[c, sl], 6), 1)
            mpar = lax.bitwise_and(
                lax.shift_right_logical(morig_v[c, sl], 6), 1)
            for r in range(LANES):
                row = rbase + r
                pu = upar[r] != 0
                pm = mpar[r] != 0
                acc = None
                for d in range(0, EMB, LANES):
                    u0 = urows_v[row, pl.ds(d, LANES)]
                    u1 = urows_v[row, pl.ds(EMB + d, LANES)]
                    m0 = mrows_v[row, pl.ds(d, LANES)]
                    m1 = mrows_v[row, pl.ds(EMB + d, LANES)]
                    uu = jnp.where(pu, u1, u0)
                    mm = jnp.where(pm, m1, m0)
                    p = uu * mm
                    acc = p if acc is None else acc + p
                buf_v[pl.ds(r * (LANES + 1), LANES)] = acc
            tot = None
            for col in range(LANES):
                v = plsc.load_gather(buf_v, [base_idx + col])
                tot = v if tot is None else tot + v
            ulane = lax.bitwise_and(uorig_v[c, sl], 127)
            mlane = lax.bitwise_and(morig_v[c, sl], 127)
            bu = plsc.load_gather(ubrows_v, [rbase + iota, ulane])
            bm = plsc.load_gather(mbrows_v, [rbase + iota, mlane])
            out_v[pl.ds(c * CHUNK + rbase, LANES)] = tot + bu + bm + gb
        return carry

    lax.fori_loop(0, NCHUNK, chunk_body, 0)

    pltpu.sync_copy(out_v, out_hbm.at[pl.ds(base, BPW)])


def kernel(user, movie, user_embedding, movie_embedding,
           user_bias_embedding, movie_bias_embedding, global_bias):
    n_user = user_embedding.shape[0]
    n_movie = movie_embedding.shape[0]
    upk = _relayout(jnp.swapaxes(user_embedding, 0, 1), n_user)
    mpk = _relayout(jnp.swapaxes(movie_embedding, 0, 1), n_movie)
    ubr = -(-n_user // PADW)
    mbr = -(-n_movie // PADW)
    ubp = jnp.pad(user_bias_embedding,
                  ((0, ubr * PADW - n_user), (0, 0))).reshape(ubr, PADW)
    mbp = jnp.pad(movie_bias_embedding,
                  ((0, mbr * PADW - n_movie), (0, 0))).reshape(mbr, PADW)
    mesh = plsc.VectorSubcoreMesh(core_axis_name="c", subcore_axis_name="s",
                                  num_cores=NC, num_subcores=NS)
    run = pl.kernel(
        _body,
        out_type=jax.ShapeDtypeStruct((BATCH,), jnp.float32),
        mesh=mesh,
        compiler_params=pltpu.CompilerParams(needs_layout_passes=False,
                                             use_tc_tiling_on_sc=True),
        scratch_types=[
            pltpu.VMEM((NCHUNK, CHUNK), jnp.int32),
            pltpu.VMEM((NCHUNK, CHUNK), jnp.int32),
            pltpu.VMEM((NCHUNK, CHUNK), jnp.int32),
            pltpu.VMEM((NCHUNK, CHUNK), jnp.int32),
            pltpu.VMEM((NCHUNK, CHUNK), jnp.int32),
            pltpu.VMEM((NCHUNK, CHUNK), jnp.int32),
            pltpu.VMEM((CHUNK, PADW), jnp.float32),
            pltpu.VMEM((CHUNK, PADW), jnp.float32),
            pltpu.VMEM((CHUNK, PADW), jnp.float32),
            pltpu.VMEM((CHUNK, PADW), jnp.float32),
            pltpu.VMEM((LANES,), jnp.float32),
            pltpu.VMEM((BPW,), jnp.float32),
            pltpu.VMEM((LANES * (LANES + 1),), jnp.float32),
            pltpu.SemaphoreType.DMA,
        ],
    )
    return run(user, movie, upk, mpk, ubp, mbp, global_bias)


# speedup vs baseline: 2.6136x; 2.6136x over previous
import functools
import jax
import jax.numpy as jnp
from jax import lax
from jax.experimental import pallas as pl
from jax.experimental.pallas import tpu as pltpu
from jax.experimental.pallas import tpu_sc as plsc

NC = 2
NS = 16
NW = NC * NS
LANES = 16
BATCH = 16384
EMB = 64
PADW = 128
BPW = BATCH // NW
CHUNK = 128
NCHUNK = BPW // CHUNK
GROUPS = CHUNK // LANES
TBLK = 2048


def _transpose_body(in_ref, eye_ref, out_ref):
    x = in_ref[...]
    t_full = lax.dot_general(x, eye_ref[...],
                             (((0,), (0,)), ((), ())),
                             preferred_element_type=jnp.float32)
    for t in range(TBLK // PADW):
        osl = pl.ds(t * EMB, EMB)
        out_ref[osl, pl.ds(0, EMB)] = t_full[t * PADW:t * PADW + EMB, :]
        out_ref[osl, pl.ds(EMB, EMB)] = (
            t_full[t * PADW + EMB:t * PADW + 2 * EMB, :])


def _relayout(table_t, n_rows):
    grid = -(-n_rows // TBLK)
    out_rows = -(-n_rows // PADW) * EMB
    eye = jnp.eye(EMB, dtype=jnp.float32)
    return pl.pallas_call(
        _transpose_body,
        grid=(grid,),
        in_specs=[
            pl.BlockSpec((EMB, TBLK), lambda i: (0, i)),
            pl.BlockSpec((EMB, EMB), lambda i: (0, 0)),
        ],
        out_specs=pl.BlockSpec((TBLK // 2, PADW), lambda i: (i, 0)),
        out_shape=jax.ShapeDtypeStruct((out_rows, PADW), jnp.float32),
    )(table_t, eye)


def _body(user_hbm, movie_hbm, uemb_hbm, memb_hbm, ubias_hbm, mbias_hbm,
          gbias_hbm, out_hbm,
          uorig_v, morig_v, upack_v, mpack_v, ubrow_v, mbrow_v,
          urows_v, mrows_v, ubrows_v, mbrows_v, gb_v, out_v, buf_v, sem):
    wid = lax.axis_index("s") * NC + lax.axis_index("c")
    base = wid * BPW

    for j in range(NCHUNK):
        pltpu.sync_copy(user_hbm.at[pl.ds(base + j * CHUNK, CHUNK)],
                        uorig_v.at[j])
        pltpu.sync_copy(movie_hbm.at[pl.ds(base + j * CHUNK, CHUNK)],
                        morig_v.at[j])
    pltpu.sync_copy(gbias_hbm, gb_v.at[pl.ds(0, 1)])

    for j in range(NCHUNK):
        for h in range(GROUPS):
            sl = pl.ds(h * LANES, LANES)
            u = uorig_v[j, sl]
            m = morig_v[j, sl]
            ub = lax.shift_right_logical(u, 7)
            mb = lax.shift_right_logical(m, 7)
            upack_v[j, sl] = lax.bitwise_or(
                lax.shift_left(ub, 6), lax.bitwise_and(u, 63))
            mpack_v[j, sl] = lax.bitwise_or(
                lax.shift_left(mb, 6), lax.bitwise_and(m, 63))
            ubrow_v[j, sl] = ub
            mbrow_v[j, sl] = mb

    gb = gb_v[pl.ds(0, LANES)][0]
    iota = jax.lax.iota(jnp.int32, LANES)
    base_idx = iota * (LANES + 1)

    def chunk_body(c, carry):
        cu = pltpu.async_copy(uemb_hbm.at[upack_v.at[c]], urows_v, sem)
        cm = pltpu.async_copy(memb_hbm.at[mpack_v.at[c]], mrows_v, sem)
        cub = pltpu.async_copy(ubias_hbm.at[ubrow_v.at[c]], ubrows_v, sem)
        cmb = pltpu.async_copy(mbias_hbm.at[mbrow_v.at[c]], mbrows_v, sem)
        cu.wait()
        cm.wait()
        cub.wait()
        cmb.wait()

        for g in range(GROUPS):
            rbase = g * LANES
            sl = pl.ds(rbase, LANES)
            upar = lax.bitwise_and(
                lax.shift_right_logical(uorig_v[c, sl], 6), 1)
            mpar = lax.bitwise_and(
                lax.shift_right_logical(morig_v[c, sl], 6), 1)
            for r in range(LANES):
                row = rbase + r
                pu = upar[r] != 0
                pm = mpar[r] != 0
                acc = None
                for d in range(0, EMB, LANES):
                    u0 = urows_v[row, pl.ds(d, LANES)]
                    u1 = urows_v[row, pl.ds(EMB + d, LANES)]
                    m0 = mrows_v[row, pl.ds(d, LANES)]
                    m1 = mrows_v[row, pl.ds(EMB + d, LANES)]
                    uu = jnp.where(pu, u1, u0)
                    mm = jnp.where(pm, m1, m0)
                    p = uu * mm
                    acc = p if acc is None else acc + p
                buf_v[pl.ds(r * (LANES + 1), LANES)] = acc
            tot = None
            for col in range(LANES):
                v = plsc.load_gather(buf_v, [base_idx + col])
                tot = v if tot is None else tot + v
            ulane = lax.bitwise_and(uorig_v[c, sl], 127)
            mlane = lax.bitwise_and(morig_v[c, sl], 127)
            bu = plsc.load_gather(ubrows_v, [rbase + iota, ulane])
            bm = plsc.load_gather(mbrows_v, [rbase + iota, mlane])
            out_v[pl.ds(c * CHUNK + rbase, LANES)] = tot + bu + bm + gb
        return carry

    lax.fori_loop(0, NCHUNK, chunk_body, 0)

    pltpu.sync_copy(out_v, out_hbm.at[pl.ds(base, BPW)])


def kernel(user, movie, user_embedding, movie_embedding,
           user_bias_embedding, movie_bias_embedding, global_bias):
    n_user = user_embedding.shape[0]
    n_movie = movie_embedding.shape[0]
    upk = _relayout(jnp.swapaxes(user_embedding, 0, 1), n_user)
    mpk = _relayout(jnp.swapaxes(movie_embedding, 0, 1), n_movie)
    ubr = -(-n_user // PADW)
    mbr = -(-n_movie // PADW)
    ubp = jnp.pad(user_bias_embedding,
                  ((0, ubr * PADW - n_user), (0, 0))).reshape(ubr, PADW)
    mbp = jnp.pad(movie_bias_embedding,
                  ((0, mbr * PADW - n_movie), (0, 0))).reshape(mbr, PADW)
    mesh = plsc.VectorSubcoreMesh(core_axis_name="c", subcore_axis_name="s",
                                  num_cores=NC, num_subcores=NS)
    run = pl.kernel(
        _body,
        out_type=jax.ShapeDtypeStruct((BATCH,), jnp.float32),
        mesh=mesh,
        compiler_params=pltpu.CompilerParams(needs_layout_passes=False,
                                             use_tc_tiling_on_sc=True),
        scratch_types=[
            pltpu.VMEM((NCHUNK, CHUNK), jnp.int32),
            pltpu.VMEM((NCHUNK, CHUNK), jnp.int32),
            pltpu.VMEM((NCHUNK, CHUNK), jnp.int32),
            pltpu.VMEM((NCHUNK, CHUNK), jnp.int32),
            pltpu.VMEM((NCHUNK, CHUNK), jnp.int32),
            pltpu.VMEM((NCHUNK, CHUNK), jnp.int32),
            pltpu.VMEM((CHUNK, PADW), jnp.float32),
            pltpu.VMEM((CHUNK, PADW), jnp.float32),
            pltpu.VMEM((CHUNK, PADW), jnp.float32),
            pltpu.VMEM((CHUNK, PADW), jnp.float32),
            pltpu.VMEM((LANES,), jnp.float32),
            pltpu.VMEM((BPW,), jnp.float32),
            pltpu.VMEM((LANES * (LANES + 1),), jnp.float32),
            pltpu.SemaphoreType.DMA,
        ],
    )
    return run(user, movie, upk, mpk, ubp, mbp, global_bias)


# speedup vs baseline: 2.6180x; 1.0017x over previous
import functools
import jax
import jax.numpy as jnp
from jax import lax
from jax.experimental import pallas as pl
from jax.experimental.pallas import tpu as pltpu
from jax.experimental.pallas import tpu_sc as plsc

NC = 2
NS = 16
NW = NC * NS
LANES = 16
BATCH = 16384
EMB = 64
PADW = 128
BPW = BATCH // NW
CHUNK = 128
NCHUNK = BPW // CHUNK
GROUPS = CHUNK // LANES
TBLK = 2048


def _transpose_body(in_ref, eye_ref, out_ref):
    x = in_ref[...]
    t_full = lax.dot_general(x, eye_ref[...],
                             (((0,), (0,)), ((), ())),
                             preferred_element_type=jnp.float32)
    for t in range(TBLK // PADW):
        osl = pl.ds(t * EMB, EMB)
        out_ref[osl, pl.ds(0, EMB)] = t_full[t * PADW:t * PADW + EMB, :]
        out_ref[osl, pl.ds(EMB, EMB)] = (
            t_full[t * PADW + EMB:t * PADW + 2 * EMB, :])


def _relayout(table_t, n_rows):
    grid = -(-n_rows // TBLK)
    out_rows = -(-n_rows // PADW) * EMB
    eye = jnp.eye(EMB, dtype=jnp.float32)
    return pl.pallas_call(
        _transpose_body,
        grid=(grid,),
        in_specs=[
            pl.BlockSpec((EMB, TBLK), lambda i: (0, i)),
            pl.BlockSpec((EMB, EMB), lambda i: (0, 0)),
        ],
        out_specs=pl.BlockSpec((TBLK // 2, PADW), lambda i: (i, 0)),
        out_shape=jax.ShapeDtypeStruct((out_rows, PADW), jnp.float32),
        compiler_params=pltpu.CompilerParams(
            fuse_transposed_lhs_in_matmul=True),
    )(table_t, eye)


def _body(user_hbm, movie_hbm, uemb_hbm, memb_hbm, ubias_hbm, mbias_hbm,
          gbias_hbm, out_hbm,
          uorig_v, morig_v, upack_v, mpack_v, ubrow_v, mbrow_v,
          urows_v, mrows_v, ubrows_v, mbrows_v, gb_v, out_v, buf_v, sem):
    wid = lax.axis_index("s") * NC + lax.axis_index("c")
    base = wid * BPW

    for j in range(NCHUNK):
        pltpu.sync_copy(user_hbm.at[pl.ds(base + j * CHUNK, CHUNK)],
                        uorig_v.at[j])
        pltpu.sync_copy(movie_hbm.at[pl.ds(base + j * CHUNK, CHUNK)],
                        morig_v.at[j])
    pltpu.sync_copy(gbias_hbm, gb_v.at[pl.ds(0, 1)])

    for j in range(NCHUNK):
        for h in range(GROUPS):
            sl = pl.ds(h * LANES, LANES)
            u = uorig_v[j, sl]
            m = morig_v[j, sl]
            ub = lax.shift_right_logical(u, 7)
            mb = lax.shift_right_logical(m, 7)
            upack_v[j, sl] = lax.bitwise_or(
                lax.shift_left(ub, 6), lax.bitwise_and(u, 63))
            mpack_v[j, sl] = lax.bitwise_or(
                lax.shift_left(mb, 6), lax.bitwise_and(m, 63))
            ubrow_v[j, sl] = ub
            mbrow_v[j, sl] = mb

    gb = gb_v[pl.ds(0, LANES)][0]
    iota = jax.lax.iota(jnp.int32, LANES)
    base_idx = iota * (LANES + 1)

    def chunk_body(c, carry):
        cu = pltpu.async_copy(uemb_hbm.at[upack_v.at[c]], urows_v, sem)
        cm = pltpu.async_copy(memb_hbm.at[mpack_v.at[c]], mrows_v, sem)
        cub = pltpu.async_copy(ubias_hbm.at[ubrow_v.at[c]], ubrows_v, sem)
        cmb = pltpu.async_copy(mbias_hbm.at[mbrow_v.at[c]], mbrows_v, sem)
        cu.wait()
        cm.wait()
        cub.wait()
        cmb.wait()

        for g in range(GROUPS):
            rbase = g * LANES
            sl = pl.ds(rbase, LANES)
            upar = lax.bitwise_and(
                lax.shift_right_logical(uorig_v[c, sl], 6), 1)
            mpar = lax.bitwise_and(
                lax.shift_right_logical(morig_v[c, sl], 6), 1)
            for r in range(LANES):
                row = rbase + r
                pu = upar[r] != 0
                pm = mpar[r] != 0
                acc = None
                for d in range(0, EMB, LANES):
                    u0 = urows_v[row, pl.ds(d, LANES)]
                    u1 = urows_v[row, pl.ds(EMB + d, LANES)]
                    m0 = mrows_v[row, pl.ds(d, LANES)]
                    m1 = mrows_v[row, pl.ds(EMB + d, LANES)]
                    uu = jnp.where(pu, u1, u0)
                    mm = jnp.where(pm, m1, m0)
                    p = uu * mm
                    acc = p if acc is None else acc + p
                buf_v[pl.ds(r * (LANES + 1), LANES)] = acc
            tot = None
            for col in range(LANES):
                v = plsc.load_gather(buf_v, [base_idx + col])
                tot = v if tot is None else tot + v
            ulane = lax.bitwise_and(uorig_v[c, sl], 127)
            mlane = lax.bitwise_and(morig_v[c, sl], 127)
            bu = plsc.load_gather(ubrows_v, [rbase + iota, ulane])
            bm = plsc.load_gather(mbrows_v, [rbase + iota, mlane])
            out_v[pl.ds(c * CHUNK + rbase, LANES)] = tot + bu + bm + gb
        return carry

    lax.fori_loop(0, NCHUNK, chunk_body, 0)

    pltpu.sync_copy(out_v, out_hbm.at[pl.ds(base, BPW)])


def kernel(user, movie, user_embedding, movie_embedding,
           user_bias_embedding, movie_bias_embedding, global_bias):
    n_user = user_embedding.shape[0]
    n_movie = movie_embedding.shape[0]
    upk = _relayout(jnp.swapaxes(user_embedding, 0, 1), n_user)
    mpk = _relayout(jnp.swapaxes(movie_embedding, 0, 1), n_movie)
    ubr = -(-n_user // PADW)
    mbr = -(-n_movie // PADW)
    ubp = jnp.pad(user_bias_embedding,
                  ((0, ubr * PADW - n_user), (0, 0))).reshape(ubr, PADW)
    mbp = jnp.pad(movie_bias_embedding,
                  ((0, mbr * PADW - n_movie), (0, 0))).reshape(mbr, PADW)
    mesh = plsc.VectorSubcoreMesh(core_axis_name="c", subcore_axis_name="s",
                                  num_cores=NC, num_subcores=NS)
    run = pl.kernel(
        _body,
        out_type=jax.ShapeDtypeStruct((BATCH,), jnp.float32),
        mesh=mesh,
        compiler_params=pltpu.CompilerParams(needs_layout_passes=False,
                                             use_tc_tiling_on_sc=True),
        scratch_types=[
            pltpu.VMEM((NCHUNK, CHUNK), jnp.int32),
            pltpu.VMEM((NCHUNK, CHUNK), jnp.int32),
            pltpu.VMEM((NCHUNK, CHUNK), jnp.int32),
            pltpu.VMEM((NCHUNK, CHUNK), jnp.int32),
            pltpu.VMEM((NCHUNK, CHUNK), jnp.int32),
            pltpu.VMEM((NCHUNK, CHUNK), jnp.int32),
            pltpu.VMEM((CHUNK, PADW), jnp.float32),
            pltpu.VMEM((CHUNK, PADW), jnp.float32),
            pltpu.VMEM((CHUNK, PADW), jnp.float32),
            pltpu.VMEM((CHUNK, PADW), jnp.float32),
            pltpu.VMEM((LANES,), jnp.float32),
            pltpu.VMEM((BPW,), jnp.float32),
            pltpu.VMEM((LANES * (LANES + 1),), jnp.float32),
            pltpu.SemaphoreType.DMA,
        ],
    )
    return run(user, movie, upk, mpk, ubp, mbp, global_bias)
